# Initial kernel scaffold; baseline (speedup 1.0000x reference)
import functools

import jax
import jax.numpy as jnp
from jax import lax
from jax.experimental import pallas as pl
from jax.experimental.pallas import tpu as pltpu
from jax.experimental.pallas import tpu_sc as plsc

N = 10000
E = 320000
D = 128
H = 32
C = 40

NC = 2
NS = 16
NW = NC * NS
K = 80
NCHUNK = E // NW // K
RPT = N // NS
DEGW = 16

_mesh = plsc.VectorSubcoreMesh(core_axis_name="c", subcore_axis_name="s")


@functools.partial(
    pl.kernel,
    out_type=jax.ShapeDtypeStruct((NC, N, DEGW), jnp.float32),
    mesh=_mesh,
    scratch_types=[
        pltpu.VMEM_SHARED((N, DEGW), jnp.float32),
        pltpu.VMEM((RPT, DEGW), jnp.float32),
        pltpu.VMEM((NCHUNK, K), jnp.int32),
    ],
)
def _deg_kernel(dst_hbm, out_hbm, acc, ones_v, dst_v):
    c = lax.axis_index("c")
    s = lax.axis_index("s")
    wid = s * NC + c
    ones16 = jnp.full((DEGW,), 1.0, dtype=jnp.float32)

    def fill(i, carry):
        ones_v[i, :] = ones16
        return carry

    lax.fori_loop(0, RPT, fill, 0)
    pltpu.sync_copy(ones_v, acc.at[pl.ds(s * RPT, RPT)])
    pltpu.sync_copy(dst_hbm.at[pl.ds(wid * NCHUNK, NCHUNK)], dst_v)
    plsc.subcore_barrier()

    def body(g, carry):
        pltpu.sync_copy(ones_v.at[pl.ds(0, K)], acc.at[dst_v.at[g]], add=True)
        return carry

    lax.fori_loop(0, NCHUNK, body, 0)
    plsc.subcore_barrier()
    pltpu.sync_copy(acc.at[pl.ds(s * RPT, RPT)], out_hbm.at[c, pl.ds(s * RPT, RPT)])


@functools.partial(
    pl.kernel,
    out_type=jax.ShapeDtypeStruct((NC, N, H), jnp.float32),
    mesh=_mesh,
    scratch_types=[
        pltpu.VMEM_SHARED((N, H), jnp.float32),
        pltpu.VMEM((NCHUNK, K), jnp.int32),
        pltpu.VMEM((NCHUNK, K), jnp.int32),
        pltpu.VMEM((K, H), jnp.float32),
        pltpu.SemaphoreType.DMA,
    ],
)
def _agg_kernel(m_hbm, src_hbm, dst_hbm, out_hbm, acc, src_v, dst_v, rows_v, sem):
    c = lax.axis_index("c")
    s = lax.axis_index("s")
    wid = s * NC + c
    pltpu.sync_copy(m_hbm.at[pl.ds(s * RPT, RPT)], acc.at[pl.ds(s * RPT, RPT)])
    pltpu.sync_copy(src_hbm.at[pl.ds(wid * NCHUNK, NCHUNK)], src_v)
    pltpu.sync_copy(dst_hbm.at[pl.ds(wid * NCHUNK, NCHUNK)], dst_v)
    plsc.subcore_barrier()

    def body(g, carry):
        pltpu.async_copy(m_hbm.at[src_v.at[g]], rows_v, sem).wait()
        pltpu.sync_copy(rows_v, acc.at[dst_v.at[g]], add=True)
        return carry

    lax.fori_loop(0, NCHUNK, body, 0)
    plsc.subcore_barrier()
    pltpu.sync_copy(acc.at[pl.ds(s * RPT, RPT)], out_hbm.at[c, pl.ds(s * RPT, RPT)])


def _pre_body(x_ref, w1_ref, d0_ref, d1_ref, m_ref, dinv_ref):
    deg = d0_ref[...] + d1_ref[...] - 1.0
    dinv = lax.rsqrt(deg)
    h = jnp.dot(x_ref[...], w1_ref[...], preferred_element_type=jnp.float32)
    m_ref[...] = h * dinv
    dinv_ref[...] = dinv


_pre_call = pl.pallas_call(
    _pre_body,
    out_shape=[
        jax.ShapeDtypeStruct((N, H), jnp.float32),
        jax.ShapeDtypeStruct((N, 1), jnp.float32),
    ],
)


def _mid1_body(a_ref, m1_ref, dinv_ref, b1_ref, w2_ref, m2_ref):
    agg = a_ref[0] + a_ref[1] - m1_ref[...]
    dinv = dinv_ref[...]
    t = jnp.maximum(agg * dinv + b1_ref[...], 0.0)
    m2_ref[...] = jnp.dot(t, w2_ref[...], preferred_element_type=jnp.float32) * dinv


_mid1_call = pl.pallas_call(
    _mid1_body,
    out_shape=jax.ShapeDtypeStruct((N, H), jnp.float32),
)


def _mid2_body(a_ref, m2_ref, dinv_ref, b2_ref, m3_ref):
    agg = a_ref[0] + a_ref[1] - m2_ref[...]
    dinv = dinv_ref[...]
    m3_ref[...] = jnp.maximum(agg * dinv + b2_ref[...], 0.0) * dinv


_mid2_call = pl.pallas_call(
    _mid2_body,
    out_shape=jax.ShapeDtypeStruct((N, H), jnp.float32),
)


def _post_body(a_ref, m3_ref, dinv_ref, b3_ref, w3_ref, o_ref):
    agg = (a_ref[0] + a_ref[1] - m3_ref[...]) * dinv_ref[...]
    o_ref[...] = jnp.dot(agg, w3_ref[...], preferred_element_type=jnp.float32) + b3_ref[...]


_post_call = pl.pallas_call(
    _post_body,
    out_shape=jax.ShapeDtypeStruct((N, C), jnp.float32),
)


def kernel(x, edge_index, W1, b1, W2, b2, W3, b3):
    src = edge_index[0].reshape(E // K, K)
    dst = edge_index[1].reshape(E // K, K)
    degp = _deg_kernel(dst)
    d0 = degp[0, :, 0:1]
    d1 = degp[1, :, 0:1]
    m1, dinv = _pre_call(x, W1, d0, d1)
    a1 = _agg_kernel(m1, src, dst)
    m2 = _mid1_call(a1, m1, dinv, b1.reshape(1, H), W2)
    a2 = _agg_kernel(m2, src, dst)
    m3 = _mid2_call(a2, m2, dinv, b2.reshape(1, H))
    a3 = _agg_kernel(m3, src, dst)
    out = _post_call(a3, m3, dinv, b3.reshape(1, C), W3)
    return out


# speedup vs baseline: 23.5878x; 23.5878x over previous
import functools

import jax
import jax.numpy as jnp
from jax import lax
from jax.experimental import pallas as pl
from jax.experimental.pallas import tpu as pltpu
from jax.experimental.pallas import tpu_sc as plsc

N = 10000
E = 320000
D = 128
H = 32
C = 40

NC = 2
NS = 16
NW = NC * NS
K = 80
NCHUNK = E // NW // K
RB = 1000
NRB = N // RB
DEGW = 16

_mesh = plsc.VectorSubcoreMesh(core_axis_name="c", subcore_axis_name="s")


@functools.partial(
    pl.kernel,
    out_type=jax.ShapeDtypeStruct((NC, N, DEGW), jnp.float32),
    mesh=_mesh,
    compiler_params=pltpu.CompilerParams(use_tc_tiling_on_sc=False),
    scratch_types=[
        pltpu.VMEM_SHARED((N, DEGW), jnp.float32),
        pltpu.VMEM((RB, DEGW), jnp.float32),
        pltpu.VMEM((NCHUNK, K), jnp.int32),
    ],
)
def _deg_kernel(dst_hbm, out_hbm, acc, ones_v, dst_v):
    c = lax.axis_index("c")
    s = lax.axis_index("s")
    wid = s * NC + c
    ones16 = jnp.full((DEGW,), 1.0, dtype=jnp.float32)

    def fill(i, carry):
        ones_v[i, :] = ones16
        return carry

    lax.fori_loop(0, RB, fill, 0)

    @pl.when(s < NRB)
    def _():
        pltpu.sync_copy(ones_v, acc.at[pl.ds(s * RB, RB)])

    pltpu.sync_copy(dst_hbm.at[wid], dst_v)
    plsc.subcore_barrier()

    def body(g, carry):
        pltpu.sync_copy(ones_v.at[pl.ds(0, K)], acc.at[dst_v.at[g]], add=True)
        return carry

    lax.fori_loop(0, NCHUNK, body, 0)
    plsc.subcore_barrier()

    @pl.when(s < NRB)
    def _():
        pltpu.sync_copy(acc.at[pl.ds(s * RB, RB)], out_hbm.at[c, pl.ds(s * RB, RB)])


@functools.partial(
    pl.kernel,
    out_type=jax.ShapeDtypeStruct((NC, N, H), jnp.float32),
    mesh=_mesh,
    compiler_params=pltpu.CompilerParams(use_tc_tiling_on_sc=False),
    scratch_types=[
        pltpu.VMEM_SHARED((N, H), jnp.float32),
        pltpu.VMEM((NCHUNK, K), jnp.int32),
        pltpu.VMEM((NCHUNK, K), jnp.int32),
        pltpu.VMEM((K, H), jnp.float32),
        pltpu.SemaphoreType.DMA,
    ],
)
def _agg_kernel(m_hbm, src_hbm, dst_hbm, out_hbm, acc, src_v, dst_v, rows_v, sem):
    c = lax.axis_index("c")
    s = lax.axis_index("s")
    wid = s * NC + c
    @pl.when(s < NRB)
    def _():
        pltpu.sync_copy(m_hbm.at[pl.ds(s * RB, RB)], acc.at[pl.ds(s * RB, RB)])

    pltpu.sync_copy(src_hbm.at[wid], src_v)
    pltpu.sync_copy(dst_hbm.at[wid], dst_v)
    plsc.subcore_barrier()

    def body(g, carry):
        pltpu.async_copy(m_hbm.at[src_v.at[g]], rows_v, sem).wait()
        pltpu.sync_copy(rows_v, acc.at[dst_v.at[g]], add=True)
        return carry

    lax.fori_loop(0, NCHUNK, body, 0)
    plsc.subcore_barrier()

    @pl.when(s < NRB)
    def _():
        pltpu.sync_copy(acc.at[pl.ds(s * RB, RB)], out_hbm.at[c, pl.ds(s * RB, RB)])


def _pre_body(x_ref, w1_ref, d0_ref, d1_ref, m_ref, dinv_ref):
    deg = d0_ref[...] + d1_ref[...] - 1.0
    dinv = lax.rsqrt(deg)
    h = jnp.dot(x_ref[...], w1_ref[...], preferred_element_type=jnp.float32)
    m_ref[...] = h * dinv
    dinv_ref[...] = dinv


_pre_call = pl.pallas_call(
    _pre_body,
    out_shape=[
        jax.ShapeDtypeStruct((N, H), jnp.float32),
        jax.ShapeDtypeStruct((N, 1), jnp.float32),
    ],
)


def _mid1_body(a_ref, m1_ref, dinv_ref, b1_ref, w2_ref, m2_ref):
    agg = a_ref[0] + a_ref[1] - m1_ref[...]
    dinv = dinv_ref[...]
    t = jnp.maximum(agg * dinv + b1_ref[...], 0.0)
    m2_ref[...] = jnp.dot(t, w2_ref[...], preferred_element_type=jnp.float32) * dinv


_mid1_call = pl.pallas_call(
    _mid1_body,
    out_shape=jax.ShapeDtypeStruct((N, H), jnp.float32),
)


def _mid2_body(a_ref, m2_ref, dinv_ref, b2_ref, m3_ref):
    agg = a_ref[0] + a_ref[1] - m2_ref[...]
    dinv = dinv_ref[...]
    m3_ref[...] = jnp.maximum(agg * dinv + b2_ref[...], 0.0) * dinv


_mid2_call = pl.pallas_call(
    _mid2_body,
    out_shape=jax.ShapeDtypeStruct((N, H), jnp.float32),
)


def _post_body(a_ref, m3_ref, dinv_ref, b3_ref, w3_ref, o_ref):
    agg = (a_ref[0] + a_ref[1] - m3_ref[...]) * dinv_ref[...]
    o_ref[...] = jnp.dot(agg, w3_ref[...], preferred_element_type=jnp.float32) + b3_ref[...]


_post_call = pl.pallas_call(
    _post_body,
    out_shape=jax.ShapeDtypeStruct((N, C), jnp.float32),
)


def kernel(x, edge_index, W1, b1, W2, b2, W3, b3):
    src = edge_index[0].reshape(NW, NCHUNK, K)
    dst = edge_index[1].reshape(NW, NCHUNK, K)
    degp = _deg_kernel(dst)
    d0 = degp[0, :, 0:1]
    d1 = degp[1, :, 0:1]
    m1, dinv = _pre_call(x, W1, d0, d1)
    a1 = _agg_kernel(m1, src, dst)
    m2 = _mid1_call(a1, m1, dinv, b1.reshape(1, H), W2)
    a2 = _agg_kernel(m2, src, dst)
    m3 = _mid2_call(a2, m2, dinv, b2.reshape(1, H))
    a3 = _agg_kernel(m3, src, dst)
    out = _post_call(a3, m3, dinv, b3.reshape(1, C), W3)
    return out


# speedup vs baseline: 35.1055x; 1.4883x over previous
import functools

import jax
import jax.numpy as jnp
from jax import lax
from jax.experimental import pallas as pl
from jax.experimental.pallas import tpu as pltpu
from jax.experimental.pallas import tpu_sc as plsc

N = 10000
E = 320000
D = 128
H = 32
C = 40

NC = 2
NS = 16
NW = NC * NS
K = 80
NCHUNK = E // NW // K
RB = 1000
NRB = N // RB
DEGW = 16

_mesh = plsc.VectorSubcoreMesh(core_axis_name="c", subcore_axis_name="s")


@functools.partial(
    pl.kernel,
    out_type=jax.ShapeDtypeStruct((NC, N, DEGW), jnp.float32),
    mesh=_mesh,
    compiler_params=pltpu.CompilerParams(use_tc_tiling_on_sc=False),
    scratch_types=[
        pltpu.VMEM_SHARED((N, DEGW), jnp.float32),
        pltpu.VMEM((RB, DEGW), jnp.float32),
        pltpu.VMEM((NCHUNK, K), jnp.int32),
        pltpu.SemaphoreType.DMA,
    ],
)
def _deg_kernel(dst_hbm, out_hbm, acc, ones_v, dst_v, sem):
    c = lax.axis_index("c")
    s = lax.axis_index("s")
    wid = s * NC + c
    ones16 = jnp.full((DEGW,), 1.0, dtype=jnp.float32)

    def fill(i, carry):
        ones_v[i, :] = ones16
        return carry

    lax.fori_loop(0, RB, fill, 0)

    @pl.when(s < NRB)
    def _():
        pltpu.sync_copy(ones_v, acc.at[pl.ds(s * RB, RB)])

    pltpu.sync_copy(dst_hbm.at[wid], dst_v)
    plsc.subcore_barrier()

    def body(g, carry):
        pltpu.async_copy(ones_v.at[pl.ds(0, K)], acc.at[dst_v.at[g]], sem, add=True)

        @pl.when(g >= 3)
        def _():
            pltpu.make_async_copy(
                ones_v.at[pl.ds(0, K)], acc.at[dst_v.at[g]], sem
            ).wait()

        return carry

    lax.fori_loop(0, NCHUNK, body, 0)
    for _tail in range(3):
        pltpu.make_async_copy(
            ones_v.at[pl.ds(0, K)], acc.at[dst_v.at[NCHUNK - 1]], sem
        ).wait()
    plsc.subcore_barrier()

    @pl.when(s < NRB)
    def _():
        pltpu.sync_copy(acc.at[pl.ds(s * RB, RB)], out_hbm.at[c, pl.ds(s * RB, RB)])


@functools.partial(
    pl.kernel,
    out_type=jax.ShapeDtypeStruct((NC, N, H), jnp.float32),
    mesh=_mesh,
    compiler_params=pltpu.CompilerParams(use_tc_tiling_on_sc=False),
    scratch_types=[
        pltpu.VMEM_SHARED((N, H), jnp.float32),
        pltpu.VMEM((NCHUNK, K), jnp.int32),
        pltpu.VMEM((NCHUNK, K), jnp.int32),
        pltpu.VMEM((K, H), jnp.float32),
        pltpu.VMEM((K, H), jnp.float32),
        pltpu.SemaphoreType.DMA,
        pltpu.SemaphoreType.DMA,
    ],
)
def _agg_kernel(m_hbm, src_hbm, dst_hbm, out_hbm, acc, src_v, dst_v,
                rows_a, rows_b, gsem_a, gsem_b):
    c = lax.axis_index("c")
    s = lax.axis_index("s")
    wid = s * NC + c
    pltpu.sync_copy(src_hbm.at[wid], src_v)
    pltpu.sync_copy(dst_hbm.at[wid], dst_v)
    pltpu.async_copy(m_hbm.at[src_v.at[0]], rows_a, gsem_a)
    pltpu.async_copy(m_hbm.at[src_v.at[1]], rows_b, gsem_b)

    @pl.when(s < NRB)
    def _():
        pltpu.sync_copy(m_hbm.at[pl.ds(s * RB, RB)], acc.at[pl.ds(s * RB, RB)])

    plsc.subcore_barrier()

    def pair(i, carry):
        g = 2 * i
        pltpu.make_async_copy(m_hbm.at[src_v.at[g]], rows_a, gsem_a).wait()
        pltpu.sync_copy(rows_a, acc.at[dst_v.at[g]], add=True)

        @pl.when(g + 2 < NCHUNK)
        def _():
            pltpu.async_copy(m_hbm.at[src_v.at[g + 2]], rows_a, gsem_a)

        pltpu.make_async_copy(m_hbm.at[src_v.at[g + 1]], rows_b, gsem_b).wait()
        pltpu.sync_copy(rows_b, acc.at[dst_v.at[g + 1]], add=True)

        @pl.when(g + 3 < NCHUNK)
        def _():
            pltpu.async_copy(m_hbm.at[src_v.at[g + 3]], rows_b, gsem_b)

        return carry

    lax.fori_loop(0, NCHUNK // 2, pair, 0)
    pltpu.make_async_copy(m_hbm.at[src_v.at[NCHUNK - 1]], rows_a, gsem_a).wait()
    pltpu.sync_copy(rows_a, acc.at[dst_v.at[NCHUNK - 1]], add=True)
    plsc.subcore_barrier()

    @pl.when(s < NRB)
    def _():
        pltpu.sync_copy(acc.at[pl.ds(s * RB, RB)], out_hbm.at[c, pl.ds(s * RB, RB)])


def _pre_body(x_ref, w1_ref, d0_ref, d1_ref, m_ref, dinv_ref):
    deg = d0_ref[...] + d1_ref[...] - 1.0
    dinv = lax.rsqrt(deg)
    h = jnp.dot(x_ref[...], w1_ref[...], preferred_element_type=jnp.float32)
    m_ref[...] = h * dinv
    dinv_ref[...] = dinv


_pre_call = pl.pallas_call(
    _pre_body,
    out_shape=[
        jax.ShapeDtypeStruct((N, H), jnp.float32),
        jax.ShapeDtypeStruct((N, 1), jnp.float32),
    ],
)


def _mid1_body(a_ref, m1_ref, dinv_ref, b1_ref, w2_ref, m2_ref):
    agg = a_ref[0] + a_ref[1] - m1_ref[...]
    dinv = dinv_ref[...]
    t = jnp.maximum(agg * dinv + b1_ref[...], 0.0)
    m2_ref[...] = jnp.dot(t, w2_ref[...], preferred_element_type=jnp.float32) * dinv


_mid1_call = pl.pallas_call(
    _mid1_body,
    out_shape=jax.ShapeDtypeStruct((N, H), jnp.float32),
)


def _mid2_body(a_ref, m2_ref, dinv_ref, b2_ref, m3_ref):
    agg = a_ref[0] + a_ref[1] - m2_ref[...]
    dinv = dinv_ref[...]
    m3_ref[...] = jnp.maximum(agg * dinv + b2_ref[...], 0.0) * dinv


_mid2_call = pl.pallas_call(
    _mid2_body,
    out_shape=jax.ShapeDtypeStruct((N, H), jnp.float32),
)


def _post_body(a_ref, m3_ref, dinv_ref, b3_ref, w3_ref, o_ref):
    agg = (a_ref[0] + a_ref[1] - m3_ref[...]) * dinv_ref[...]
    o_ref[...] = jnp.dot(agg, w3_ref[...], preferred_element_type=jnp.float32) + b3_ref[...]


_post_call = pl.pallas_call(
    _post_body,
    out_shape=jax.ShapeDtypeStruct((N, C), jnp.float32),
)


def kernel(x, edge_index, W1, b1, W2, b2, W3, b3):
    src = edge_index[0].reshape(NW, NCHUNK, K)
    dst = edge_index[1].reshape(NW, NCHUNK, K)
    degp = _deg_kernel(dst)
    d0 = degp[0, :, 0:1]
    d1 = degp[1, :, 0:1]
    m1, dinv = _pre_call(x, W1, d0, d1)
    a1 = _agg_kernel(m1, src, dst)
    m2 = _mid1_call(a1, m1, dinv, b1.reshape(1, H), W2)
    a2 = _agg_kernel(m2, src, dst)
    m3 = _mid2_call(a2, m2, dinv, b2.reshape(1, H))
    a3 = _agg_kernel(m3, src, dst)
    out = _post_call(a3, m3, dinv, b3.reshape(1, C), W3)
    return out


# speedup vs baseline: 45.2860x; 1.2900x over previous
import functools

import jax
import jax.numpy as jnp
from jax import lax
from jax.experimental import pallas as pl
from jax.experimental.pallas import tpu as pltpu
from jax.experimental.pallas import tpu_sc as plsc

N = 10000
E = 320000
D = 128
H = 32
C = 40

NC = 2
NS = 16
NW = NC * NS
K = 80
NCHUNK = E // NW // K
RB = 1000
NRB = N // RB
DEGW = 16

_mesh = plsc.VectorSubcoreMesh(core_axis_name="c", subcore_axis_name="s")


@functools.partial(
    pl.kernel,
    out_type=jax.ShapeDtypeStruct((NC, N, DEGW), jnp.float32),
    mesh=_mesh,
    compiler_params=pltpu.CompilerParams(use_tc_tiling_on_sc=False),
    scratch_types=[
        pltpu.VMEM_SHARED((N, DEGW), jnp.float32),
        pltpu.VMEM((RB, DEGW), jnp.float32),
        pltpu.VMEM((NCHUNK, K), jnp.int32),
        pltpu.SemaphoreType.DMA,
    ],
)
def _deg_kernel(dst_hbm, out_hbm, acc, ones_v, dst_v, sem):
    c = lax.axis_index("c")
    s = lax.axis_index("s")
    wid = s * NC + c
    ones16 = jnp.full((DEGW,), 1.0, dtype=jnp.float32)

    def fill(i, carry):
        ones_v[i, :] = ones16
        return carry

    lax.fori_loop(0, RB, fill, 0)

    @pl.when(s < NRB)
    def _():
        pltpu.sync_copy(ones_v, acc.at[pl.ds(s * RB, RB)])

    pltpu.sync_copy(dst_hbm.at[wid], dst_v)
    plsc.subcore_barrier()

    def body(g, carry):
        pltpu.async_copy(ones_v.at[pl.ds(0, K)], acc.at[dst_v.at[g]], sem, add=True)

        @pl.when(g >= 3)
        def _():
            pltpu.make_async_copy(
                ones_v.at[pl.ds(0, K)], acc.at[dst_v.at[g]], sem
            ).wait()

        return carry

    lax.fori_loop(0, NCHUNK, body, 0)
    for _tail in range(3):
        pltpu.make_async_copy(
            ones_v.at[pl.ds(0, K)], acc.at[dst_v.at[NCHUNK - 1]], sem
        ).wait()
    plsc.subcore_barrier()

    @pl.when(s < NRB)
    def _():
        pltpu.sync_copy(acc.at[pl.ds(s * RB, RB)], out_hbm.at[c, pl.ds(s * RB, RB)])


@functools.partial(
    pl.kernel,
    out_type=jax.ShapeDtypeStruct((NC, N, H), jnp.float32),
    mesh=_mesh,
    compiler_params=pltpu.CompilerParams(use_tc_tiling_on_sc=False),
    scratch_types=[
        pltpu.VMEM_SHARED((N, H), jnp.float32),
        pltpu.VMEM((NCHUNK, K), jnp.int32),
        pltpu.VMEM((NCHUNK, K), jnp.int32),
        pltpu.VMEM((K, H), jnp.float32),
        pltpu.VMEM((K, H), jnp.float32),
        pltpu.VMEM((K, H), jnp.float32),
        pltpu.VMEM((K, H), jnp.float32),
        pltpu.VMEM((K, H), jnp.float32),
        pltpu.SemaphoreType.DMA,
        pltpu.SemaphoreType.DMA,
        pltpu.SemaphoreType.DMA,
        pltpu.SemaphoreType.DMA,
        pltpu.SemaphoreType.DMA,
        pltpu.SemaphoreType.DMA,
        pltpu.SemaphoreType.DMA,
        pltpu.SemaphoreType.DMA,
        pltpu.SemaphoreType.DMA,
        pltpu.SemaphoreType.DMA,
    ],
)
def _agg_kernel(m_hbm, src_hbm, dst_hbm, out_hbm, acc, src_v, dst_v,
                r0, r1, r2, r3, r4,
                g0, g1, g2, g3, g4,
                s0, s1, s2, s3, s4):
    c = lax.axis_index("c")
    s = lax.axis_index("s")
    wid = s * NC + c
    rows = (r0, r1, r2, r3, r4)
    gsem = (g0, g1, g2, g3, g4)
    ssem = (s0, s1, s2, s3, s4)
    NB = 5
    pltpu.sync_copy(src_hbm.at[wid], src_v)
    pltpu.sync_copy(dst_hbm.at[wid], dst_v)
    for b in range(NB):
        pltpu.async_copy(m_hbm.at[src_v.at[b]], rows[b], gsem[b])

    @pl.when(s < NRB)
    def _():
        pltpu.sync_copy(m_hbm.at[pl.ds(s * RB, RB)], acc.at[pl.ds(s * RB, RB)])

    plsc.subcore_barrier()

    def step(i, carry):
        g = NB * i
        for b in range(NB):
            pltpu.make_async_copy(m_hbm.at[src_v.at[g + b]], rows[b], gsem[b]).wait()
            pltpu.async_copy(rows[b], acc.at[dst_v.at[g + b]], ssem[b], add=True)
        for b in range(NB):
            @pl.when(g + b + NB < NCHUNK)
            def _(b=b):
                pltpu.make_async_copy(rows[b], acc.at[dst_v.at[g + b]], ssem[b]).wait()
                pltpu.async_copy(m_hbm.at[src_v.at[g + b + NB]], rows[b], gsem[b])

        return carry

    lax.fori_loop(0, NCHUNK // NB, step, 0)
    for b in range(NB):
        pltpu.make_async_copy(rows[b], acc.at[dst_v.at[NCHUNK - NB + b]], ssem[b]).wait()
    plsc.subcore_barrier()

    @pl.when(s < NRB)
    def _():
        pltpu.sync_copy(acc.at[pl.ds(s * RB, RB)], out_hbm.at[c, pl.ds(s * RB, RB)])


def _pre_body(x_ref, w1_ref, d0_ref, d1_ref, m_ref, dinv_ref):
    deg = d0_ref[...] + d1_ref[...] - 1.0
    dinv = lax.rsqrt(deg)
    h = jnp.dot(x_ref[...], w1_ref[...], preferred_element_type=jnp.float32)
    m_ref[...] = h * dinv
    dinv_ref[...] = dinv


_pre_call = pl.pallas_call(
    _pre_body,
    out_shape=[
        jax.ShapeDtypeStruct((N, H), jnp.float32),
        jax.ShapeDtypeStruct((N, 1), jnp.float32),
    ],
)


def _mid1_body(a_ref, m1_ref, dinv_ref, b1_ref, w2_ref, m2_ref):
    agg = a_ref[0] + a_ref[1] - m1_ref[...]
    dinv = dinv_ref[...]
    t = jnp.maximum(agg * dinv + b1_ref[...], 0.0)
    m2_ref[...] = jnp.dot(t, w2_ref[...], preferred_element_type=jnp.float32) * dinv


_mid1_call = pl.pallas_call(
    _mid1_body,
    out_shape=jax.ShapeDtypeStruct((N, H), jnp.float32),
)


def _mid2_body(a_ref, m2_ref, dinv_ref, b2_ref, m3_ref):
    agg = a_ref[0] + a_ref[1] - m2_ref[...]
    dinv = dinv_ref[...]
    m3_ref[...] = jnp.maximum(agg * dinv + b2_ref[...], 0.0) * dinv


_mid2_call = pl.pallas_call(
    _mid2_body,
    out_shape=jax.ShapeDtypeStruct((N, H), jnp.float32),
)


def _post_body(a_ref, m3_ref, dinv_ref, b3_ref, w3_ref, o_ref):
    agg = (a_ref[0] + a_ref[1] - m3_ref[...]) * dinv_ref[...]
    o_ref[...] = jnp.dot(agg, w3_ref[...], preferred_element_type=jnp.float32) + b3_ref[...]


_post_call = pl.pallas_call(
    _post_body,
    out_shape=jax.ShapeDtypeStruct((N, C), jnp.float32),
)


def kernel(x, edge_index, W1, b1, W2, b2, W3, b3):
    src = edge_index[0].reshape(NW, NCHUNK, K)
    dst = edge_index[1].reshape(NW, NCHUNK, K)
    degp = _deg_kernel(dst)
    d0 = degp[0, :, 0:1]
    d1 = degp[1, :, 0:1]
    m1, dinv = _pre_call(x, W1, d0, d1)
    a1 = _agg_kernel(m1, src, dst)
    m2 = _mid1_call(a1, m1, dinv, b1.reshape(1, H), W2)
    a2 = _agg_kernel(m2, src, dst)
    m3 = _mid2_call(a2, m2, dinv, b2.reshape(1, H))
    a3 = _agg_kernel(m3, src, dst)
    out = _post_call(a3, m3, dinv, b3.reshape(1, C), W3)
    return out


# speedup vs baseline: 57.1841x; 1.2627x over previous
import functools

import jax
import jax.numpy as jnp
from jax import lax
from jax.experimental import pallas as pl
from jax.experimental.pallas import tpu as pltpu
from jax.experimental.pallas import tpu_sc as plsc

N = 10000
E = 320000
D = 128
H = 32
C = 40

NC = 2
NS = 16
NW = NC * NS
K = 80
NCHUNK = E // NW // K
RB = 1000
NRB = N // RB
DEGW = 32

_mesh = plsc.VectorSubcoreMesh(core_axis_name="c", subcore_axis_name="s")


@functools.partial(
    pl.kernel,
    out_type=jax.ShapeDtypeStruct((NC, N, DEGW), jnp.float32),
    mesh=_mesh,
    compiler_params=pltpu.CompilerParams(use_tc_tiling_on_sc=False),
    scratch_types=[
        pltpu.VMEM_SHARED((N, DEGW), jnp.float32),
        pltpu.VMEM((RB, DEGW), jnp.float32),
        pltpu.VMEM((NCHUNK, K), jnp.int32),
        pltpu.SemaphoreType.DMA,
    ],
)
def _deg_kernel(dst_hbm, out_hbm, acc, ones_v, dst_v, sem):
    c = lax.axis_index("c")
    s = lax.axis_index("s")
    wid = s * NC + c
    ones16 = jnp.full((DEGW,), 1.0, dtype=jnp.float32)

    def fill(i, carry):
        ones_v[i, :] = ones16
        return carry

    lax.fori_loop(0, RB, fill, 0)

    @pl.when(s < NRB)
    def _():
        pltpu.sync_copy(ones_v, acc.at[pl.ds(s * RB, RB)])

    pltpu.sync_copy(dst_hbm.at[wid], dst_v)
    plsc.subcore_barrier()

    def body(g, carry):
        pltpu.async_copy(ones_v.at[pl.ds(0, K)], acc.at[dst_v.at[g]], sem, add=True)

        @pl.when(g >= 3)
        def _():
            pltpu.make_async_copy(
                ones_v.at[pl.ds(0, K)], acc.at[dst_v.at[g]], sem
            ).wait()

        return carry

    lax.fori_loop(0, NCHUNK, body, 0)
    for _tail in range(3):
        pltpu.make_async_copy(
            ones_v.at[pl.ds(0, K)], acc.at[dst_v.at[NCHUNK - 1]], sem
        ).wait()
    plsc.subcore_barrier()

    @pl.when(s < NRB)
    def _():
        pltpu.sync_copy(acc.at[pl.ds(s * RB, RB)], out_hbm.at[c, pl.ds(s * RB, RB)])


@functools.partial(
    pl.kernel,
    out_type=jax.ShapeDtypeStruct((NC, N, H), jnp.float32),
    mesh=_mesh,
    compiler_params=pltpu.CompilerParams(use_tc_tiling_on_sc=False),
    scratch_types=[
        pltpu.VMEM_SHARED((N, H), jnp.float32),
        pltpu.VMEM((NCHUNK, K), jnp.int32),
        pltpu.VMEM((NCHUNK, K), jnp.int32),
        pltpu.VMEM((K, H), jnp.float32),
        pltpu.VMEM((K, H), jnp.float32),
        pltpu.VMEM((K, H), jnp.float32),
        pltpu.VMEM((K, H), jnp.float32),
        pltpu.VMEM((K, H), jnp.float32),
        pltpu.SemaphoreType.DMA,
        pltpu.SemaphoreType.DMA,
        pltpu.SemaphoreType.DMA,
        pltpu.SemaphoreType.DMA,
        pltpu.SemaphoreType.DMA,
        pltpu.SemaphoreType.DMA,
        pltpu.SemaphoreType.DMA,
        pltpu.SemaphoreType.DMA,
        pltpu.SemaphoreType.DMA,
        pltpu.SemaphoreType.DMA,
    ],
)
def _agg_kernel(m_hbm, src_hbm, dst_hbm, out_hbm, acc, src_v, dst_v,
                r0, r1, r2, r3, r4,
                g0, g1, g2, g3, g4,
                s0, s1, s2, s3, s4):
    c = lax.axis_index("c")
    s = lax.axis_index("s")
    wid = s * NC + c
    rows = (r0, r1, r2, r3, r4)
    gsem = (g0, g1, g2, g3, g4)
    ssem = (s0, s1, s2, s3, s4)
    NB = 5
    pltpu.sync_copy(src_hbm.at[wid], src_v)
    pltpu.sync_copy(dst_hbm.at[wid], dst_v)
    for b in range(NB):
        pltpu.async_copy(m_hbm.at[src_v.at[b]], rows[b], gsem[b])

    @pl.when(s < NRB)
    def _():
        pltpu.sync_copy(m_hbm.at[pl.ds(s * RB, RB)], acc.at[pl.ds(s * RB, RB)])

    plsc.subcore_barrier()

    def step(i, carry):
        g = NB * i
        for b in range(NB):
            pltpu.make_async_copy(m_hbm.at[src_v.at[g + b]], rows[b], gsem[b]).wait()
            pltpu.async_copy(rows[b], acc.at[dst_v.at[g + b]], ssem[b], add=True)
        for b in range(NB):
            @pl.when(g + b + NB < NCHUNK)
            def _(b=b):
                pltpu.make_async_copy(rows[b], acc.at[dst_v.at[g + b]], ssem[b]).wait()
                pltpu.async_copy(m_hbm.at[src_v.at[g + b + NB]], rows[b], gsem[b])

        return carry

    lax.fori_loop(0, NCHUNK // NB, step, 0)
    for b in range(NB):
        pltpu.make_async_copy(rows[b], acc.at[dst_v.at[NCHUNK - NB + b]], ssem[b]).wait()
    plsc.subcore_barrier()

    @pl.when(s < NRB)
    def _():
        pltpu.sync_copy(acc.at[pl.ds(s * RB, RB)], out_hbm.at[c, pl.ds(s * RB, RB)])


def _pre_body(x4_ref, w1d_ref, d_ref, m_ref, dinv_ref):
    dinv = lax.rsqrt(d_ref[0] + d_ref[1] - 1.0)
    m_ref[...] = jnp.dot(x4_ref[...], w1d_ref[...],
                         preferred_element_type=jnp.float32) * dinv
    dinv_ref[...] = dinv


_pre_call = pl.pallas_call(
    _pre_body,
    out_shape=[
        jax.ShapeDtypeStruct((N // 4, 128), jnp.float32),
        jax.ShapeDtypeStruct((N // 4, 128), jnp.float32),
    ],
)


def _mid1_body(a_ref, m1_ref, dinv_ref, b1_ref, w2d_ref, m2_ref):
    dinv = dinv_ref[...]
    t = jnp.maximum((a_ref[0] + a_ref[1] - m1_ref[...]) * dinv + b1_ref[...], 0.0)
    m2_ref[...] = jnp.dot(t, w2d_ref[...], preferred_element_type=jnp.float32) * dinv


_mid1_call = pl.pallas_call(
    _mid1_body,
    out_shape=jax.ShapeDtypeStruct((N // 4, 128), jnp.float32),
)


def _mid2_body(a_ref, m2_ref, dinv_ref, b2_ref, m3_ref):
    dinv = dinv_ref[...]
    m3_ref[...] = jnp.maximum(
        (a_ref[0] + a_ref[1] - m2_ref[...]) * dinv + b2_ref[...], 0.0) * dinv


_mid2_call = pl.pallas_call(
    _mid2_body,
    out_shape=jax.ShapeDtypeStruct((N // 4, 128), jnp.float32),
)


def _post_body(a_ref, m3_ref, dinv_ref, b3_ref, w3d_ref, o_ref):
    sf = (a_ref[0] + a_ref[1] - m3_ref[...]) * dinv_ref[...]
    o_ref[...] = jnp.dot(sf, w3d_ref[...],
                         preferred_element_type=jnp.float32) + b3_ref[...]


_post_call = pl.pallas_call(
    _post_body,
    out_shape=jax.ShapeDtypeStruct((N // 4, 4 * C), jnp.float32),
)


def _block_diag4(w):
    a, b = w.shape
    out = jnp.zeros((4 * a, 4 * b), dtype=w.dtype)
    for q in range(4):
        out = out.at[q * a:(q + 1) * a, q * b:(q + 1) * b].set(w)
    return out


def kernel(x, edge_index, W1, b1, W2, b2, W3, b3):
    src = edge_index[0].reshape(NW, NCHUNK, K)
    dst = edge_index[1].reshape(NW, NCHUNK, K)
    x4 = x.reshape(N // 4, 4 * D)
    w1d = _block_diag4(W1)
    w2d = _block_diag4(W2)
    w3d = _block_diag4(W3)
    b1t = jnp.tile(b1, 4).reshape(1, 128)
    b2t = jnp.tile(b2, 4).reshape(1, 128)
    b3t = jnp.tile(b3, 4).reshape(1, 4 * C)

    degp = _deg_kernel(dst).reshape(NC, N // 4, 128)
    m1f, dinvf = _pre_call(x4, w1d, degp)
    a1 = _agg_kernel(m1f.reshape(N, H), src, dst).reshape(NC, N // 4, 128)
    m2f = _mid1_call(a1, m1f, dinvf, b1t, w2d)
    a2 = _agg_kernel(m2f.reshape(N, H), src, dst).reshape(NC, N // 4, 128)
    m3f = _mid2_call(a2, m2f, dinvf, b2t)
    a3 = _agg_kernel(m3f.reshape(N, H), src, dst).reshape(NC, N // 4, 128)
    out = _post_call(a3, m3f, dinvf, b3t, w3d)
    return out.reshape(N, C)


# speedup vs baseline: 65.1984x; 1.1401x over previous
import functools

import jax
import jax.numpy as jnp
from jax import lax
from jax.experimental import pallas as pl
from jax.experimental.pallas import tpu as pltpu
from jax.experimental.pallas import tpu_sc as plsc

N = 10000
E = 320000
D = 128
H = 32
C = 40

NC = 2
NS = 16
NW = NC * NS
K = 128
NCHUNK = E // K
CPW = NCHUNK // NW
NXTRA = NCHUNK - CPW * NW
NB = 6
RB = 1000
NRB = N // RB
DEGW = 32

_mesh = plsc.VectorSubcoreMesh(core_axis_name="c", subcore_axis_name="s")


@functools.partial(
    pl.kernel,
    out_type=jax.ShapeDtypeStruct((NC, N, DEGW), jnp.float32),
    mesh=_mesh,
    compiler_params=pltpu.CompilerParams(use_tc_tiling_on_sc=False),
    scratch_types=[
        pltpu.VMEM_SHARED((N, DEGW), jnp.float32),
        pltpu.VMEM((RB, DEGW), jnp.float32),
        pltpu.VMEM((CPW, 2, K), jnp.int32),
        pltpu.VMEM((1, 2, K), jnp.int32),
        pltpu.SemaphoreType.DMA,
    ],
)
def _deg_kernel(ei_hbm, out_hbm, acc, ones_v, ei_v, eix_v, sem):
    c = lax.axis_index("c")
    s = lax.axis_index("s")
    wid = s * NC + c
    ones_row = jnp.full((DEGW,), 1.0, dtype=jnp.float32)

    def fill(i, carry):
        ones_v[i, :] = ones_row
        return carry

    lax.fori_loop(0, RB, fill, 0)

    @pl.when(s < NRB)
    def _():
        pltpu.sync_copy(ones_v, acc.at[pl.ds(s * RB, RB)])

    pltpu.sync_copy(ei_hbm.at[pl.ds(wid * CPW, CPW)], ei_v)

    @pl.when(wid < NXTRA)
    def _():
        pltpu.sync_copy(ei_hbm.at[pl.ds(NW * CPW + wid, 1)], eix_v)

    plsc.subcore_barrier()

    def body(g, carry):
        pltpu.async_copy(ones_v.at[pl.ds(0, K)], acc.at[ei_v.at[g, 1]], sem, add=True)

        @pl.when(g >= 3)
        def _():
            pltpu.make_async_copy(
                ones_v.at[pl.ds(0, K)], acc.at[ei_v.at[g, 1]], sem
            ).wait()

        return carry

    lax.fori_loop(0, CPW, body, 0)

    @pl.when(wid < NXTRA)
    def _():
        pltpu.async_copy(ones_v.at[pl.ds(0, K)], acc.at[eix_v.at[0, 1]], sem, add=True)
        pltpu.make_async_copy(ones_v.at[pl.ds(0, K)], acc.at[eix_v.at[0, 1]], sem).wait()

    for _tail in range(3):
        pltpu.make_async_copy(
            ones_v.at[pl.ds(0, K)], acc.at[ei_v.at[CPW - 1, 1]], sem
        ).wait()
    plsc.subcore_barrier()

    @pl.when(s < NRB)
    def _():
        pltpu.sync_copy(acc.at[pl.ds(s * RB, RB)], out_hbm.at[c, pl.ds(s * RB, RB)])


@functools.partial(
    pl.kernel,
    out_type=jax.ShapeDtypeStruct((NC, N, H), jnp.float32),
    mesh=_mesh,
    compiler_params=pltpu.CompilerParams(use_tc_tiling_on_sc=False),
    scratch_types=[
        pltpu.VMEM_SHARED((N, H), jnp.float32),
        pltpu.VMEM((CPW, 2, K), jnp.int32),
        pltpu.VMEM((1, 2, K), jnp.int32),
        pltpu.VMEM((K, H), jnp.float32),
        pltpu.VMEM((K, H), jnp.float32),
        pltpu.VMEM((K, H), jnp.float32),
        pltpu.VMEM((K, H), jnp.float32),
        pltpu.VMEM((K, H), jnp.float32),
        pltpu.VMEM((K, H), jnp.float32),
        pltpu.SemaphoreType.DMA,
        pltpu.SemaphoreType.DMA,
        pltpu.SemaphoreType.DMA,
        pltpu.SemaphoreType.DMA,
        pltpu.SemaphoreType.DMA,
        pltpu.SemaphoreType.DMA,
        pltpu.SemaphoreType.DMA,
        pltpu.SemaphoreType.DMA,
        pltpu.SemaphoreType.DMA,
        pltpu.SemaphoreType.DMA,
        pltpu.SemaphoreType.DMA,
        pltpu.SemaphoreType.DMA,
    ],
)
def _agg_kernel(m_hbm, ei_hbm, out_hbm, acc, ei_v, eix_v,
                r0, r1, r2, r3, r4, r5,
                g0, g1, g2, g3, g4, g5,
                s0, s1, s2, s3, s4, s5):
    c = lax.axis_index("c")
    s = lax.axis_index("s")
    wid = s * NC + c
    rows = (r0, r1, r2, r3, r4, r5)
    gsem = (g0, g1, g2, g3, g4, g5)
    ssem = (s0, s1, s2, s3, s4, s5)
    pltpu.sync_copy(ei_hbm.at[pl.ds(wid * CPW, CPW)], ei_v)

    @pl.when(wid < NXTRA)
    def _():
        pltpu.sync_copy(ei_hbm.at[pl.ds(NW * CPW + wid, 1)], eix_v)

    for b in range(NB):
        pltpu.async_copy(m_hbm.at[ei_v.at[b, 0]], rows[b], gsem[b])

    @pl.when(s < NRB)
    def _():
        pltpu.sync_copy(m_hbm.at[pl.ds(s * RB, RB)], acc.at[pl.ds(s * RB, RB)])

    plsc.subcore_barrier()

    def step(i, carry):
        g = NB * i
        for b in range(NB):
            pltpu.make_async_copy(m_hbm.at[ei_v.at[g + b, 0]], rows[b], gsem[b]).wait()
            pltpu.async_copy(rows[b], acc.at[ei_v.at[g + b, 1]], ssem[b], add=True)
        for b in range(NB):
            @pl.when(g + b + NB < CPW)
            def _(b=b):
                pltpu.make_async_copy(rows[b], acc.at[ei_v.at[g + b, 1]], ssem[b]).wait()
                pltpu.async_copy(m_hbm.at[ei_v.at[g + b + NB, 0]], rows[b], gsem[b])

        return carry

    lax.fori_loop(0, CPW // NB, step, 0)
    for b in range(NB):
        pltpu.make_async_copy(rows[b], acc.at[ei_v.at[CPW - NB + b, 1]], ssem[b]).wait()

    @pl.when(wid < NXTRA)
    def _():
        pltpu.async_copy(m_hbm.at[eix_v.at[0, 0]], rows[0], gsem[0])
        pltpu.make_async_copy(m_hbm.at[eix_v.at[0, 0]], rows[0], gsem[0]).wait()
        pltpu.async_copy(rows[0], acc.at[eix_v.at[0, 1]], ssem[0], add=True)
        pltpu.make_async_copy(rows[0], acc.at[eix_v.at[0, 1]], ssem[0]).wait()

    plsc.subcore_barrier()

    @pl.when(s < NRB)
    def _():
        pltpu.sync_copy(acc.at[pl.ds(s * RB, RB)], out_hbm.at[c, pl.ds(s * RB, RB)])


def _pre_body(x4_ref, w1d_ref, d_ref, m_ref, dinv_ref):
    dinv = lax.rsqrt(d_ref[0] + d_ref[1] - 1.0)
    m_ref[...] = jnp.dot(x4_ref[...], w1d_ref[...],
                         preferred_element_type=jnp.float32) * dinv
    dinv_ref[...] = dinv


_pre_call = pl.pallas_call(
    _pre_body,
    out_shape=[
        jax.ShapeDtypeStruct((N // 4, 128), jnp.float32),
        jax.ShapeDtypeStruct((N // 4, 128), jnp.float32),
    ],
)


def _mid1_body(a_ref, m1_ref, dinv_ref, b1_ref, w2d_ref, m2_ref):
    dinv = dinv_ref[...]
    t = jnp.maximum((a_ref[0] + a_ref[1] - m1_ref[...]) * dinv + b1_ref[...], 0.0)
    m2_ref[...] = jnp.dot(t, w2d_ref[...], preferred_element_type=jnp.float32) * dinv


_mid1_call = pl.pallas_call(
    _mid1_body,
    out_shape=jax.ShapeDtypeStruct((N // 4, 128), jnp.float32),
)


def _mid2_body(a_ref, m2_ref, dinv_ref, b2_ref, m3_ref):
    dinv = dinv_ref[...]
    m3_ref[...] = jnp.maximum(
        (a_ref[0] + a_ref[1] - m2_ref[...]) * dinv + b2_ref[...], 0.0) * dinv


_mid2_call = pl.pallas_call(
    _mid2_body,
    out_shape=jax.ShapeDtypeStruct((N // 4, 128), jnp.float32),
)


def _post_body(a_ref, m3_ref, dinv_ref, b3_ref, w3d_ref, o_ref):
    sf = (a_ref[0] + a_ref[1] - m3_ref[...]) * dinv_ref[...]
    o_ref[...] = jnp.dot(sf, w3d_ref[...],
                         preferred_element_type=jnp.float32) + b3_ref[...]


_post_call = pl.pallas_call(
    _post_body,
    out_shape=jax.ShapeDtypeStruct((N // 4, 4 * C), jnp.float32),
)


def _block_diag4(w):
    a, b = w.shape
    r = jnp.arange(4 * a) // a
    col = jnp.arange(4 * b) // b
    return jnp.where(r[:, None] == col[None, :], jnp.tile(w, (4, 4)), 0.0)


def kernel(x, edge_index, W1, b1, W2, b2, W3, b3):
    ei = edge_index.reshape(2, NCHUNK, K).transpose(1, 0, 2)
    x4 = x.reshape(N // 4, 4 * D)
    w1d = _block_diag4(W1)
    w2d = _block_diag4(W2)
    w3d = _block_diag4(W3)
    b1t = jnp.tile(b1, 4).reshape(1, 128)
    b2t = jnp.tile(b2, 4).reshape(1, 128)
    b3t = jnp.tile(b3, 4).reshape(1, 4 * C)

    degp = _deg_kernel(ei).reshape(NC, N // 4, 128)
    m1f, dinvf = _pre_call(x4, w1d, degp)
    a1 = _agg_kernel(m1f.reshape(N, H), ei).reshape(NC, N // 4, 128)
    m2f = _mid1_call(a1, m1f, dinvf, b1t, w2d)
    a2 = _agg_kernel(m2f.reshape(N, H), ei).reshape(NC, N // 4, 128)
    m3f = _mid2_call(a2, m2f, dinvf, b2t)
    a3 = _agg_kernel(m3f.reshape(N, H), ei).reshape(NC, N // 4, 128)
    out = _post_call(a3, m3f, dinvf, b3t, w3d)
    return out.reshape(N, C)
